# baseline (speedup 1.0000x reference)
import jax
import jax.numpy as jnp
from jax import lax
from jax.experimental import pallas as pl
from jax.experimental.pallas import tpu as pltpu
from jax.experimental.pallas import tpu_sc as plsc

DIM_ = 1024
NSTEPS_ = 3072
ROW_BLOCK = 512
NCORES = 2
NSUB = 16
COLS_PER = DIM_ // NSUB
STEPS_PER = NSTEPS_ // NCORES


def _cs_kernel(a_ref, o_ref):
    a = a_ref[...]
    o_ref[0, :, :] = jnp.cos(a)
    o_ref[1, :, :] = jnp.sin(a)


def _sc_build_body(pi_hbm, pj_hbm, cos_hbm, sin_hbm, out_hbm,
                   pi_v, pj_v, cos_v, sin_v, m0, m1, m2, m3):
    cid = lax.axis_index("c")
    sid = lax.axis_index("s")
    base = cid * STEPS_PER
    col0 = sid * COLS_PER
    slabs = (m0, m1, m2, m3)

    pltpu.sync_copy(pi_hbm.at[pl.ds(base, STEPS_PER)], pi_v)
    pltpu.sync_copy(pj_hbm.at[pl.ds(base, STEPS_PER)], pj_v)
    pltpu.sync_copy(cos_hbm, cos_v)
    pltpu.sync_copy(sin_hbm, sin_v)

    zeros = jnp.zeros((16,), jnp.float32)

    def zero_row(r, _):
        for m in slabs:
            m[r, :] = zeros
        return 0

    lax.fori_loop(0, DIM_, zero_row, 0)
    lanes = lax.iota(jnp.int32, 16)
    for q in range(COLS_PER):
        onehot = jnp.where(lanes == (q % 16), 1.0, 0.0).astype(jnp.float32)
        slabs[q // 16][col0 + q, :] = onehot

    def chunk_body(tc, _):
        t0 = tc * 16
        tm0 = lax.rem(base + t0, DIM_)
        pi_c = pi_v[pl.ds(t0, 16)]
        pj_c = pj_v[pl.ds(t0, 16)]
        cos_c = cos_v[pl.ds(tm0, 16)]
        sin_c = sin_v[pl.ds(tm0, 16)]
        for u in range(16):
            i = pi_c[u]
            j = pj_c[u]
            c = cos_c[u]
            s = sin_c[u]
            for m in slabs:
                mi = m[i, :]
                mj = m[j, :]
                m[i, :] = c * mi - s * mj
                m[j, :] = s * mi + c * mj
        return 0

    lax.fori_loop(0, STEPS_PER // 16, chunk_body, 0)

    for k, m in enumerate(slabs):
        pltpu.sync_copy(m, out_hbm.at[cid, :, pl.ds(col0 + 16 * k, 16)])


def _combine_kernel(a_ref, b_ref, o_ref):
    o_ref[...] = jnp.dot(
        b_ref[...], a_ref[...], preferred_element_type=jnp.float32
    )


def _matmul_kernel(x_ref, m_ref, o_ref):
    o_ref[...] = jax.lax.dot_general(
        x_ref[...], m_ref[...],
        dimension_numbers=(((1,), (1,)), ((), ())),
        preferred_element_type=jnp.float32,
    )


def kernel(x, angles, pairs_i, pairs_j):
    dim = angles.shape[0]
    x2d = x.reshape(-1, dim).astype(jnp.float32)
    n_rows = x2d.shape[0]

    cs = pl.pallas_call(
        _cs_kernel,
        out_shape=jax.ShapeDtypeStruct((2, 8, 128), jnp.float32),
    )(angles.reshape(8, 128).astype(jnp.float32))
    cs = cs.reshape(2, dim)

    mesh = plsc.VectorSubcoreMesh(
        core_axis_name="c", subcore_axis_name="s",
        num_cores=NCORES, num_subcores=NSUB,
    )
    sc_build = pl.kernel(
        _sc_build_body,
        out_type=jax.ShapeDtypeStruct((NCORES, dim, dim), jnp.float32),
        mesh=mesh,
        scratch_types=[
            pltpu.VMEM((STEPS_PER,), jnp.int32),
            pltpu.VMEM((STEPS_PER,), jnp.int32),
            pltpu.VMEM((dim,), jnp.float32),
            pltpu.VMEM((dim,), jnp.float32),
            pltpu.VMEM((dim, 16), jnp.float32),
            pltpu.VMEM((dim, 16), jnp.float32),
            pltpu.VMEM((dim, 16), jnp.float32),
            pltpu.VMEM((dim, 16), jnp.float32),
        ],
        compiler_params=pltpu.CompilerParams(use_tc_tiling_on_sc=False),
    )
    halves = sc_build(pairs_i, pairs_j, cs[0], cs[1])

    c = pl.pallas_call(
        _combine_kernel,
        out_shape=jax.ShapeDtypeStruct((dim, dim), jnp.float32),
    )(halves[0], halves[1])

    grid = (n_rows // ROW_BLOCK,)
    y2d = pl.pallas_call(
        _matmul_kernel,
        out_shape=jax.ShapeDtypeStruct((n_rows, dim), jnp.float32),
        grid=grid,
        in_specs=[
            pl.BlockSpec((ROW_BLOCK, dim), lambda r: (r, 0)),
            pl.BlockSpec((dim, dim), lambda r: (0, 0)),
        ],
        out_specs=pl.BlockSpec((ROW_BLOCK, dim), lambda r: (r, 0)),
    )(x2d, c)

    return y2d.reshape(x.shape).astype(x.dtype)


# speedup vs baseline: 1.2405x; 1.2405x over previous
import jax
import jax.numpy as jnp
from jax.experimental import pallas as pl
from jax.experimental.pallas import tpu as pltpu

DIM_ = 1024
ROW_BLOCK = 512
NCHUNK = 4
NSTEPS_ = 3072
CHUNK_STEPS = NSTEPS_ // NCHUNK


def _build_m_kernel(pairs_i_ref, pairs_j_ref, angles_ref, *refs):
    m_refs = refs[:NCHUNK]
    cs_ref = refs[NCHUNK]

    a = angles_ref[...]
    cs_ref[:, 0:1] = jnp.cos(a)
    cs_ref[:, 1:2] = jnp.sin(a)

    p_ids = jax.lax.broadcasted_iota(jnp.int32, (DIM_ * 8, 128), 0)
    l_ids = jax.lax.broadcasted_iota(jnp.int32, (DIM_ * 8, 128), 1)
    logical_col = 128 * (p_ids % 8) + l_ids
    eye8 = jnp.where(logical_col == p_ids // 8, 1.0, 0.0).astype(jnp.float32)
    for m_ref in m_refs:
        m_ref[...] = eye8

    def body(t, _):
        for q, m_ref in enumerate(m_refs):
            tq = q * CHUNK_STEPS + t
            ib = pairs_i_ref[tq] * 8
            jb = pairs_j_ref[tq] * 8
            tm = jax.lax.rem(tq, DIM_)
            c = cs_ref[pl.ds(tm, 1), 0:1]
            s = cs_ref[pl.ds(tm, 1), 1:2]
            mi = m_ref[pl.ds(ib, 8), :]
            mj = m_ref[pl.ds(jb, 8), :]
            m_ref[pl.ds(ib, 8), :] = c * mi - s * mj
            m_ref[pl.ds(jb, 8), :] = s * mi + c * mj
        return 0

    jax.lax.fori_loop(0, CHUNK_STEPS, body, 0, unroll=2)


def _combine_kernel(*refs):
    ms = [jnp.reshape(r[...], (DIM_, DIM_)) for r in refs[:NCHUNK]]
    o_ref = refs[NCHUNK]
    acc = ms[0]
    for q in range(1, NCHUNK):
        acc = jnp.dot(ms[q], acc, preferred_element_type=jnp.float32)
    o_ref[...] = acc


def _matmul_kernel(x_ref, m_ref, o_ref):
    o_ref[...] = jax.lax.dot_general(
        x_ref[...], m_ref[...],
        dimension_numbers=(((1,), (1,)), ((), ())),
        preferred_element_type=jnp.float32,
    )


def kernel(x, angles, pairs_i, pairs_j):
    dim = angles.shape[0]
    x2d = x.reshape(-1, dim).astype(jnp.float32)
    n_rows = x2d.shape[0]

    m8s = pl.pallas_call(
        _build_m_kernel,
        out_shape=[
            jax.ShapeDtypeStruct((dim * 8, 128), jnp.float32)
        ] * NCHUNK,
        in_specs=[
            pl.BlockSpec(memory_space=pltpu.SMEM),
            pl.BlockSpec(memory_space=pltpu.SMEM),
            pl.BlockSpec(memory_space=pltpu.VMEM),
        ],
        out_specs=[pl.BlockSpec(memory_space=pltpu.VMEM)] * NCHUNK,
        scratch_shapes=[pltpu.VMEM((dim, 2), jnp.float32)],
    )(pairs_i, pairs_j, angles.reshape(dim, 1).astype(jnp.float32))

    m = pl.pallas_call(
        _combine_kernel,
        out_shape=jax.ShapeDtypeStruct((dim, dim), jnp.float32),
    )(*m8s)

    grid = (n_rows // ROW_BLOCK,)
    y2d = pl.pallas_call(
        _matmul_kernel,
        out_shape=jax.ShapeDtypeStruct((n_rows, dim), jnp.float32),
        grid=grid,
        in_specs=[
            pl.BlockSpec((ROW_BLOCK, dim), lambda r: (r, 0)),
            pl.BlockSpec((dim, dim), lambda r: (0, 0)),
        ],
        out_specs=pl.BlockSpec((ROW_BLOCK, dim), lambda r: (r, 0)),
    )(x2d, m)

    return y2d.reshape(x.shape).astype(x.dtype)


# speedup vs baseline: 1.3988x; 1.1276x over previous
import jax
import jax.numpy as jnp
from jax.experimental import pallas as pl
from jax.experimental.pallas import tpu as pltpu

DIM_ = 1024
ROW_BLOCK = 2048


def _build_m_kernel(pairs_i_ref, pairs_j_ref, angles_ref, m_ref, cs_ref):
    a = angles_ref[...]
    cs_ref[:, 0:1] = jnp.cos(a)
    cs_ref[:, 1:2] = jnp.sin(a)

    p_ids = jax.lax.broadcasted_iota(jnp.int32, (DIM_ * 8, 128), 0)
    l_ids = jax.lax.broadcasted_iota(jnp.int32, (DIM_ * 8, 128), 1)
    logical_col = 128 * (p_ids % 8) + l_ids
    m_ref[...] = jnp.where(logical_col == p_ids // 8, 1.0, 0.0).astype(
        jnp.float32
    )

    n_steps = pairs_i_ref.shape[0]

    def body(t, _):
        ib = pairs_i_ref[t] * 8
        jb = pairs_j_ref[t] * 8
        tm = jax.lax.rem(t, DIM_)
        c = cs_ref[pl.ds(tm, 1), 0:1]
        s = cs_ref[pl.ds(tm, 1), 1:2]
        mi = m_ref[pl.ds(ib, 8), :]
        mj = m_ref[pl.ds(jb, 8), :]
        m_ref[pl.ds(ib, 8), :] = c * mi - s * mj
        m_ref[pl.ds(jb, 8), :] = s * mi + c * mj
        return 0

    jax.lax.fori_loop(0, n_steps, body, 0, unroll=8)


def _matmul_kernel(x_ref, m_ref, o_ref):
    o_ref[...] = jax.lax.dot_general(
        x_ref[...], m_ref[...],
        dimension_numbers=(((1,), (1,)), ((), ())),
        preferred_element_type=jnp.float32,
    )


def kernel(x, angles, pairs_i, pairs_j):
    dim = angles.shape[0]
    x2d = x.reshape(-1, dim).astype(jnp.float32)
    n_rows = x2d.shape[0]

    m8 = pl.pallas_call(
        _build_m_kernel,
        out_shape=jax.ShapeDtypeStruct((dim * 8, 128), jnp.float32),
        in_specs=[
            pl.BlockSpec(memory_space=pltpu.SMEM),
            pl.BlockSpec(memory_space=pltpu.SMEM),
            pl.BlockSpec(memory_space=pltpu.VMEM),
        ],
        out_specs=pl.BlockSpec(memory_space=pltpu.VMEM),
        scratch_shapes=[pltpu.VMEM((dim, 2), jnp.float32)],
    )(pairs_i, pairs_j, angles.reshape(dim, 1).astype(jnp.float32))
    m = m8.reshape(dim, dim)

    grid = (n_rows // ROW_BLOCK,)
    y2d = pl.pallas_call(
        _matmul_kernel,
        out_shape=jax.ShapeDtypeStruct((n_rows, dim), jnp.float32),
        grid=grid,
        in_specs=[
            pl.BlockSpec((ROW_BLOCK, dim), lambda r: (r, 0)),
            pl.BlockSpec((dim, dim), lambda r: (0, 0)),
        ],
        out_specs=pl.BlockSpec((ROW_BLOCK, dim), lambda r: (r, 0)),
    )(x2d, m)

    return y2d.reshape(x.shape).astype(x.dtype)


# speedup vs baseline: 1.7158x; 1.2266x over previous
import jax
import jax.numpy as jnp
from jax.experimental import pallas as pl
from jax.experimental.pallas import tpu as pltpu

DIM_ = 1024
ROW_BLOCK = 2048


def _build_m_kernel(pairs_i_ref, pairs_j_ref, angles_ref, m_ref, cs_ref):
    a = angles_ref[...]
    cs_ref[:, 0:1] = jnp.cos(a)
    cs_ref[:, 1:2] = jnp.sin(a)

    p_ids = jax.lax.broadcasted_iota(jnp.int32, (DIM_ * 8, 128), 0)
    l_ids = jax.lax.broadcasted_iota(jnp.int32, (DIM_ * 8, 128), 1)
    logical_col = 128 * (p_ids % 8) + l_ids
    m_ref[...] = jnp.where(logical_col == p_ids // 8, 1.0, 0.0).astype(
        jnp.float32
    )

    n_steps = pairs_i_ref.shape[0]

    def body(t, _):
        ib = pairs_i_ref[t] * 8
        jb = pairs_j_ref[t] * 8
        tm = jax.lax.rem(t, DIM_)
        c = cs_ref[pl.ds(tm, 1), 0:1]
        s = cs_ref[pl.ds(tm, 1), 1:2]
        mi = m_ref[pl.ds(ib, 8), :]
        mj = m_ref[pl.ds(jb, 8), :]
        m_ref[pl.ds(ib, 8), :] = c * mi - s * mj
        m_ref[pl.ds(jb, 8), :] = s * mi + c * mj
        return 0

    jax.lax.fori_loop(0, n_steps, body, 0, unroll=16)


def _matmul_kernel(x_ref, m_ref, o_ref):
    o_ref[...] = jax.lax.dot_general(
        x_ref[...], m_ref[...],
        dimension_numbers=(((1,), (1,)), ((), ())),
        preferred_element_type=jnp.float32,
    )


def kernel(x, angles, pairs_i, pairs_j):
    dim = angles.shape[0]
    x2d = x.reshape(-1, dim).astype(jnp.float32)
    n_rows = x2d.shape[0]

    m8 = pl.pallas_call(
        _build_m_kernel,
        out_shape=jax.ShapeDtypeStruct((dim * 8, 128), jnp.float32),
        in_specs=[
            pl.BlockSpec(memory_space=pltpu.SMEM),
            pl.BlockSpec(memory_space=pltpu.SMEM),
            pl.BlockSpec(memory_space=pltpu.VMEM),
        ],
        out_specs=pl.BlockSpec(memory_space=pltpu.VMEM),
        scratch_shapes=[pltpu.VMEM((dim, 2), jnp.float32)],
    )(pairs_i, pairs_j, angles.reshape(dim, 1).astype(jnp.float32))
    m = m8.reshape(dim, dim)

    grid = (n_rows // ROW_BLOCK,)
    y2d = pl.pallas_call(
        _matmul_kernel,
        out_shape=jax.ShapeDtypeStruct((n_rows, dim), jnp.float32),
        grid=grid,
        in_specs=[
            pl.BlockSpec((ROW_BLOCK, dim), lambda r: (r, 0)),
            pl.BlockSpec((dim, dim), lambda r: (0, 0)),
        ],
        out_specs=pl.BlockSpec((ROW_BLOCK, dim), lambda r: (r, 0)),
    )(x2d, m)

    return y2d.reshape(x.shape).astype(x.dtype)


# speedup vs baseline: 1.8996x; 1.1071x over previous
import jax
import jax.numpy as jnp
from jax.experimental import pallas as pl
from jax.experimental.pallas import tpu as pltpu

DIM_ = 1024
ROW_BLOCK = 2048


def _build_m_kernel(pairs_i_ref, pairs_j_ref, angles_ref, m_ref, cs_ref):
    a = angles_ref[...]
    cs_ref[:, 0:1] = jnp.cos(a)
    cs_ref[:, 1:2] = jnp.sin(a)

    p_ids = jax.lax.broadcasted_iota(jnp.int32, (DIM_ * 8, 128), 0)
    l_ids = jax.lax.broadcasted_iota(jnp.int32, (DIM_ * 8, 128), 1)
    logical_col = 128 * (p_ids % 8) + l_ids
    m_ref[...] = jnp.where(logical_col == p_ids // 8, 1.0, 0.0).astype(
        jnp.float32
    )

    n_steps = pairs_i_ref.shape[0]

    def body(t, _):
        ib = pairs_i_ref[t] * 8
        jb = pairs_j_ref[t] * 8
        tm = jax.lax.rem(t, DIM_)
        c = cs_ref[pl.ds(tm, 1), 0:1]
        s = cs_ref[pl.ds(tm, 1), 1:2]
        mi = m_ref[pl.ds(ib, 8), :]
        mj = m_ref[pl.ds(jb, 8), :]
        m_ref[pl.ds(ib, 8), :] = c * mi - s * mj
        m_ref[pl.ds(jb, 8), :] = s * mi + c * mj
        return 0

    jax.lax.fori_loop(0, n_steps, body, 0, unroll=32)


def _matmul_kernel(x_ref, m_ref, o_ref):
    o_ref[...] = jax.lax.dot_general(
        x_ref[...], m_ref[...],
        dimension_numbers=(((1,), (1,)), ((), ())),
        preferred_element_type=jnp.float32,
    )


def kernel(x, angles, pairs_i, pairs_j):
    dim = angles.shape[0]
    x2d = x.reshape(-1, dim).astype(jnp.float32)
    n_rows = x2d.shape[0]

    m8 = pl.pallas_call(
        _build_m_kernel,
        out_shape=jax.ShapeDtypeStruct((dim * 8, 128), jnp.float32),
        in_specs=[
            pl.BlockSpec(memory_space=pltpu.SMEM),
            pl.BlockSpec(memory_space=pltpu.SMEM),
            pl.BlockSpec(memory_space=pltpu.VMEM),
        ],
        out_specs=pl.BlockSpec(memory_space=pltpu.VMEM),
        scratch_shapes=[pltpu.VMEM((dim, 2), jnp.float32)],
    )(pairs_i, pairs_j, angles.reshape(dim, 1).astype(jnp.float32))
    m = m8.reshape(dim, dim)

    grid = (n_rows // ROW_BLOCK,)
    y2d = pl.pallas_call(
        _matmul_kernel,
        out_shape=jax.ShapeDtypeStruct((n_rows, dim), jnp.float32),
        grid=grid,
        in_specs=[
            pl.BlockSpec((ROW_BLOCK, dim), lambda r: (r, 0)),
            pl.BlockSpec((dim, dim), lambda r: (0, 0)),
        ],
        out_specs=pl.BlockSpec((ROW_BLOCK, dim), lambda r: (r, 0)),
    )(x2d, m)

    return y2d.reshape(x.shape).astype(x.dtype)


# speedup vs baseline: 1.9718x; 1.0380x over previous
import jax
import jax.numpy as jnp
from jax.experimental import pallas as pl
from jax.experimental.pallas import tpu as pltpu

DIM_ = 1024
ROW_BLOCK = 2048


def _build_m_kernel(pairs_i_ref, pairs_j_ref, angles_ref, m_ref, cs_ref):
    a = angles_ref[...]
    cs_ref[:, 0:1] = jnp.cos(a)
    cs_ref[:, 1:2] = jnp.sin(a)

    p_ids = jax.lax.broadcasted_iota(jnp.int32, (DIM_ * 8, 128), 0)
    l_ids = jax.lax.broadcasted_iota(jnp.int32, (DIM_ * 8, 128), 1)
    logical_col = 128 * (p_ids % 8) + l_ids
    m_ref[...] = jnp.where(logical_col == p_ids // 8, 1.0, 0.0).astype(
        jnp.float32
    )

    n_steps = pairs_i_ref.shape[0]

    def body(t, _):
        ib = pairs_i_ref[t] * 8
        jb = pairs_j_ref[t] * 8
        tm = jax.lax.rem(t, DIM_)
        c = cs_ref[pl.ds(tm, 1), 0:1]
        s = cs_ref[pl.ds(tm, 1), 1:2]
        mi = m_ref[pl.ds(ib, 8), :]
        mj = m_ref[pl.ds(jb, 8), :]
        m_ref[pl.ds(ib, 8), :] = c * mi - s * mj
        m_ref[pl.ds(jb, 8), :] = s * mi + c * mj
        return 0

    jax.lax.fori_loop(0, n_steps, body, 0, unroll=64)


def _matmul_kernel(x_ref, m_ref, o_ref):
    o_ref[...] = jax.lax.dot_general(
        x_ref[...], m_ref[...],
        dimension_numbers=(((1,), (1,)), ((), ())),
        preferred_element_type=jnp.float32,
    )


def kernel(x, angles, pairs_i, pairs_j):
    dim = angles.shape[0]
    x2d = x.reshape(-1, dim).astype(jnp.float32)
    n_rows = x2d.shape[0]

    m8 = pl.pallas_call(
        _build_m_kernel,
        out_shape=jax.ShapeDtypeStruct((dim * 8, 128), jnp.float32),
        in_specs=[
            pl.BlockSpec(memory_space=pltpu.SMEM),
            pl.BlockSpec(memory_space=pltpu.SMEM),
            pl.BlockSpec(memory_space=pltpu.VMEM),
        ],
        out_specs=pl.BlockSpec(memory_space=pltpu.VMEM),
        scratch_shapes=[pltpu.VMEM((dim, 2), jnp.float32)],
    )(pairs_i, pairs_j, angles.reshape(dim, 1).astype(jnp.float32))
    m = m8.reshape(dim, dim)

    grid = (n_rows // ROW_BLOCK,)
    y2d = pl.pallas_call(
        _matmul_kernel,
        out_shape=jax.ShapeDtypeStruct((n_rows, dim), jnp.float32),
        grid=grid,
        in_specs=[
            pl.BlockSpec((ROW_BLOCK, dim), lambda r: (r, 0)),
            pl.BlockSpec((dim, dim), lambda r: (0, 0)),
        ],
        out_specs=pl.BlockSpec((ROW_BLOCK, dim), lambda r: (r, 0)),
    )(x2d, m)

    return y2d.reshape(x.shape).astype(x.dtype)


# speedup vs baseline: 2.0120x; 1.0204x over previous
import jax
import jax.numpy as jnp
from jax.experimental import pallas as pl
from jax.experimental.pallas import tpu as pltpu

DIM_ = 1024
ROW_BLOCK = 2048


def _build_m_kernel(pairs_i_ref, pairs_j_ref, angles_ref, m_ref, cs_ref):
    a = angles_ref[...]
    cs_ref[:, 0:1] = jnp.cos(a)
    cs_ref[:, 1:2] = jnp.sin(a)

    p_ids = jax.lax.broadcasted_iota(jnp.int32, (DIM_ * 8, 128), 0)
    l_ids = jax.lax.broadcasted_iota(jnp.int32, (DIM_ * 8, 128), 1)
    logical_col = 128 * (p_ids % 8) + l_ids
    m_ref[...] = jnp.where(logical_col == p_ids // 8, 1.0, 0.0).astype(
        jnp.float32
    )

    n_steps = pairs_i_ref.shape[0]

    def body(t, _):
        ib = pairs_i_ref[t] * 8
        jb = pairs_j_ref[t] * 8
        tm = jax.lax.rem(t, DIM_)
        c = cs_ref[pl.ds(tm, 1), 0:1]
        s = cs_ref[pl.ds(tm, 1), 1:2]
        mi = m_ref[pl.ds(ib, 8), :]
        mj = m_ref[pl.ds(jb, 8), :]
        m_ref[pl.ds(ib, 8), :] = c * mi - s * mj
        m_ref[pl.ds(jb, 8), :] = s * mi + c * mj
        return 0

    jax.lax.fori_loop(0, n_steps, body, 0, unroll=128)


def _matmul_kernel(x_ref, m_ref, o_ref):
    o_ref[...] = jax.lax.dot_general(
        x_ref[...], m_ref[...],
        dimension_numbers=(((1,), (1,)), ((), ())),
        preferred_element_type=jnp.float32,
    )


def kernel(x, angles, pairs_i, pairs_j):
    dim = angles.shape[0]
    x2d = x.reshape(-1, dim).astype(jnp.float32)
    n_rows = x2d.shape[0]

    m8 = pl.pallas_call(
        _build_m_kernel,
        out_shape=jax.ShapeDtypeStruct((dim * 8, 128), jnp.float32),
        in_specs=[
            pl.BlockSpec(memory_space=pltpu.SMEM),
            pl.BlockSpec(memory_space=pltpu.SMEM),
            pl.BlockSpec(memory_space=pltpu.VMEM),
        ],
        out_specs=pl.BlockSpec(memory_space=pltpu.VMEM),
        scratch_shapes=[pltpu.VMEM((dim, 2), jnp.float32)],
    )(pairs_i, pairs_j, angles.reshape(dim, 1).astype(jnp.float32))
    m = m8.reshape(dim, dim)

    grid = (n_rows // ROW_BLOCK,)
    y2d = pl.pallas_call(
        _matmul_kernel,
        out_shape=jax.ShapeDtypeStruct((n_rows, dim), jnp.float32),
        grid=grid,
        in_specs=[
            pl.BlockSpec((ROW_BLOCK, dim), lambda r: (r, 0)),
            pl.BlockSpec((dim, dim), lambda r: (0, 0)),
        ],
        out_specs=pl.BlockSpec((ROW_BLOCK, dim), lambda r: (r, 0)),
    )(x2d, m)

    return y2d.reshape(x.shape).astype(x.dtype)


# speedup vs baseline: 2.0421x; 1.0149x over previous
import jax
import jax.numpy as jnp
from jax.experimental import pallas as pl
from jax.experimental.pallas import tpu as pltpu

DIM_ = 1024
ROW_BLOCK = 2048


def _build_m_kernel(pairs_i_ref, pairs_j_ref, angles_ref, m_ref, cs_ref):
    a = angles_ref[...]
    cs_ref[:, 0:1] = jnp.cos(a)
    cs_ref[:, 1:2] = jnp.sin(a)

    p_ids = jax.lax.broadcasted_iota(jnp.int32, (DIM_ * 8, 128), 0)
    l_ids = jax.lax.broadcasted_iota(jnp.int32, (DIM_ * 8, 128), 1)
    logical_col = 128 * (p_ids % 8) + l_ids
    m_ref[...] = jnp.where(logical_col == p_ids // 8, 1.0, 0.0).astype(
        jnp.float32
    )

    n_steps = pairs_i_ref.shape[0]

    def body(t, _):
        ib = pairs_i_ref[t] * 8
        jb = pairs_j_ref[t] * 8
        tm = jax.lax.rem(t, DIM_)
        c = cs_ref[pl.ds(tm, 1), 0:1]
        s = cs_ref[pl.ds(tm, 1), 1:2]
        mi = m_ref[pl.ds(ib, 8), :]
        mj = m_ref[pl.ds(jb, 8), :]
        m_ref[pl.ds(ib, 8), :] = c * mi - s * mj
        m_ref[pl.ds(jb, 8), :] = s * mi + c * mj
        return 0

    jax.lax.fori_loop(0, n_steps, body, 0, unroll=256)


def _matmul_kernel(x_ref, m_ref, o_ref):
    o_ref[...] = jax.lax.dot_general(
        x_ref[...], m_ref[...],
        dimension_numbers=(((1,), (1,)), ((), ())),
        preferred_element_type=jnp.float32,
    )


def kernel(x, angles, pairs_i, pairs_j):
    dim = angles.shape[0]
    x2d = x.reshape(-1, dim).astype(jnp.float32)
    n_rows = x2d.shape[0]

    m8 = pl.pallas_call(
        _build_m_kernel,
        out_shape=jax.ShapeDtypeStruct((dim * 8, 128), jnp.float32),
        in_specs=[
            pl.BlockSpec(memory_space=pltpu.SMEM),
            pl.BlockSpec(memory_space=pltpu.SMEM),
            pl.BlockSpec(memory_space=pltpu.VMEM),
        ],
        out_specs=pl.BlockSpec(memory_space=pltpu.VMEM),
        scratch_shapes=[pltpu.VMEM((dim, 2), jnp.float32)],
    )(pairs_i, pairs_j, angles.reshape(dim, 1).astype(jnp.float32))
    m = m8.reshape(dim, dim)

    grid = (n_rows // ROW_BLOCK,)
    y2d = pl.pallas_call(
        _matmul_kernel,
        out_shape=jax.ShapeDtypeStruct((n_rows, dim), jnp.float32),
        grid=grid,
        in_specs=[
            pl.BlockSpec((ROW_BLOCK, dim), lambda r: (r, 0)),
            pl.BlockSpec((dim, dim), lambda r: (0, 0)),
        ],
        out_specs=pl.BlockSpec((ROW_BLOCK, dim), lambda r: (r, 0)),
    )(x2d, m)

    return y2d.reshape(x.shape).astype(x.dtype)
